# Initial kernel scaffold; baseline (speedup 1.0000x reference)
import jax
import jax.numpy as jnp
from jax import lax
from jax.experimental import pallas as pl
from jax.experimental.pallas import tpu as pltpu
from jax.experimental.pallas import tpu_sc as plsc

EMBED_DIM = 64
WINDOW = 128


def _sc_gather(x_flat, table):
    n = x_flat.shape[0]
    d = table.shape[1]
    idx2d = x_flat.reshape(1, n)
    mesh = plsc.VectorSubcoreMesh(core_axis_name="core",
                                  subcore_axis_name="subcore")

    @pl.kernel(out_type=jax.ShapeDtypeStruct((n, d), table.dtype), mesh=mesh)
    def gather_kernel(x_hbm, i_hbm, o_hbm):
        def body(i_vmem, o_vmem):
            pltpu.sync_copy(x_hbm.at[i_vmem.at[0]], o_vmem)

        pltpu.emit_pipeline(
            body,
            grid=(n // WINDOW,),
            in_specs=[pl.BlockSpec((1, WINDOW), index_map=lambda i: (0, i))],
            out_specs=[pl.BlockSpec((WINDOW, d), index_map=lambda i: (i, 0))],
            core_axis_name=("core", "subcore"),
            dimension_semantics=(pltpu.PARALLEL,),
        )(i_hbm, o_hbm)

    return gather_kernel(table, idx2d)


def kernel(x, table, conv_w, conv_b, gene):
    batch, seq = x.shape
    d = table.shape[1]

    def gene_branch(operands):
        x_, table_, conv_w_, conv_b_ = operands
        flat = x_.reshape(-1).astype(jnp.int32)
        rows = _sc_gather(flat, table_)
        return rows.reshape(batch, seq, d)

    def conv_branch(operands):
        x_, table_, conv_w_, conv_b_ = operands
        xf = x_.astype(jnp.float32)
        return xf[:, :, None] * conv_w_.reshape(1, 1, d) + conv_b_.reshape(1, 1, d)

    return lax.cond(gene != 0, gene_branch, conv_branch,
                    (x, table, conv_w, conv_b))


# speedup vs baseline: 3.8786x; 3.8786x over previous
import jax
import jax.numpy as jnp
from jax import lax
from jax.experimental import pallas as pl
from jax.experimental.pallas import tpu as pltpu
from jax.experimental.pallas import tpu_sc as plsc

EMBED_DIM = 64
WINDOW = 128


def _sc_gather(x_flat, table):
    n = x_flat.shape[0]
    d = table.shape[1]
    idx2d = x_flat.reshape(1, n)
    mesh = plsc.VectorSubcoreMesh(core_axis_name="core",
                                  subcore_axis_name="subcore")

    @pl.kernel(out_type=jax.ShapeDtypeStruct((n, d), table.dtype), mesh=mesh,
               compiler_params=pltpu.CompilerParams(use_tc_tiling_on_sc=False))
    def gather_kernel(x_hbm, i_hbm, o_hbm):
        def body(i_vmem, o_vmem):
            pltpu.sync_copy(x_hbm.at[i_vmem.at[0]], o_vmem)

        pltpu.emit_pipeline(
            body,
            grid=(n // WINDOW,),
            in_specs=[pl.BlockSpec((1, WINDOW), index_map=lambda i: (0, i))],
            out_specs=[pl.BlockSpec((WINDOW, d), index_map=lambda i: (i, 0))],
            core_axis_name=("core", "subcore"),
            dimension_semantics=(pltpu.PARALLEL,),
        )(i_hbm, o_hbm)

    return gather_kernel(table, idx2d)


def kernel(x, table, conv_w, conv_b, gene):
    batch, seq = x.shape
    d = table.shape[1]

    def gene_branch(operands):
        x_, table_, conv_w_, conv_b_ = operands
        flat = x_.reshape(-1).astype(jnp.int32)
        rows = _sc_gather(flat, table_)
        return rows.reshape(batch, seq, d)

    def conv_branch(operands):
        x_, table_, conv_w_, conv_b_ = operands
        xf = x_.astype(jnp.float32)
        return xf[:, :, None] * conv_w_.reshape(1, 1, d) + conv_b_.reshape(1, 1, d)

    return lax.cond(gene != 0, gene_branch, conv_branch,
                    (x, table, conv_w, conv_b))


# speedup vs baseline: 4.3405x; 1.1191x over previous
import jax
import jax.numpy as jnp
from jax import lax
from jax.experimental import pallas as pl
from jax.experimental.pallas import tpu as pltpu
from jax.experimental.pallas import tpu_sc as plsc

EMBED_DIM = 64
WINDOW = 128
K = 4


def _sc_gather(x_flat, table):
    n = x_flat.shape[0]
    d = table.shape[1]
    idx2d = x_flat.reshape(n // WINDOW, WINDOW)
    mesh = plsc.VectorSubcoreMesh(core_axis_name="core",
                                  subcore_axis_name="subcore")

    @pl.kernel(out_type=jax.ShapeDtypeStruct((n, d), table.dtype), mesh=mesh,
               scratch_types=[pltpu.SemaphoreType.DMA],
               compiler_params=pltpu.CompilerParams(use_tc_tiling_on_sc=False))
    def gather_kernel(x_hbm, i_hbm, o_hbm, sem):
        def body(i_vmem, o_vmem):
            copies = [
                pltpu.make_async_copy(x_hbm.at[i_vmem.at[j]],
                                      o_vmem.at[pl.ds(j * WINDOW, WINDOW)],
                                      sem)
                for j in range(K)
            ]
            for c in copies:
                c.start()
            for c in copies:
                c.wait()

        pltpu.emit_pipeline(
            body,
            grid=(n // (K * WINDOW),),
            in_specs=[pl.BlockSpec((K, WINDOW), index_map=lambda i: (i, 0))],
            out_specs=[pl.BlockSpec((K * WINDOW, d), index_map=lambda i: (i, 0))],
            core_axis_name=("core", "subcore"),
            dimension_semantics=(pltpu.PARALLEL,),
        )(i_hbm, o_hbm)

    return gather_kernel(table, idx2d)


def kernel(x, table, conv_w, conv_b, gene):
    batch, seq = x.shape
    d = table.shape[1]

    def gene_branch(operands):
        x_, table_, conv_w_, conv_b_ = operands
        flat = x_.reshape(-1).astype(jnp.int32)
        rows = _sc_gather(flat, table_)
        return rows.reshape(batch, seq, d)

    def conv_branch(operands):
        x_, table_, conv_w_, conv_b_ = operands
        xf = x_.astype(jnp.float32)
        return xf[:, :, None] * conv_w_.reshape(1, 1, d) + conv_b_.reshape(1, 1, d)

    return lax.cond(gene != 0, gene_branch, conv_branch,
                    (x, table, conv_w, conv_b))
